# baseline (speedup 1.0000x reference)
import functools

import numpy as np
import jax
import jax.numpy as jnp
from jax import lax
from jax.experimental import pallas as pl
from jax.experimental.pallas import tpu as pltpu
from jax.experimental.pallas import tpu_sc as plsc

_L = 16
_F = 2
_T = 2 ** 19
_N = 131072
_HID = 64
_MASK = _T - 1

_min_res = np.array([16.0, 16.0, 16.0, 16.0])
_max_res = np.array([256.0, 256.0, 256.0, 128.0])
_b = np.exp((np.log(_max_res) - np.log(_min_res)) / (_L - 1))
_RES = np.floor(_min_res[None, :] * (_b[None, :] ** np.arange(_L)[:, None])).astype(np.int64)
_RM1 = _RES.astype(np.float32) - 1.0
_PRIMES = [int(np.int32(np.uint32(p))) for p in (1, 2654435761, 805459861, 3674653429)]

_NTILES = 32
_PTS_PER_TILE = _N // _NTILES
_CHUNK = 256
_CHUNKS_PER_TILE = _PTS_PER_TILE // _CHUNK
_NG = _CHUNK // 16
_NROWS = _CHUNK * 16
_KROWS = _NROWS // 128


def _enc_body(x_hbm, tbl_hbm, rsp_hbm, out_hbm, xcr, xc, rs, ibuf, wbuf, rows, fbuf, sem):
    cid = lax.axis_index("c")
    sid = lax.axis_index("s")
    wid = cid * 16 + sid
    pltpu.sync_copy(rsp_hbm, rs)
    iota = lax.iota(jnp.int32, 16)
    zi = jnp.zeros((16,), jnp.int32)

    def chunk_body(ch, carry):
        gchunk = wid * _CHUNKS_PER_TILE + ch
        pt0 = gchunk * _CHUNK
        pltpu.sync_copy(x_hbm.at[pl.ds(pt0, _CHUNK)], xcr)

        def tr_body(g, ct):
            g16 = g * 16
            ridx = iota + g16
            for d in range(4):
                xc[d, pl.ds(g16, 16)] = plsc.load_gather(xcr, [ridx, zi + d])
            return ct

        lax.fori_loop(0, _NG, tr_body, 0)

        def level_body(l, carry2):
            lbase = l * _T
            rm = [rs[l, d, :] for d in range(4)]

            def idx_body(g, c3):
                g16 = g * 16
                kk = g >> 3
                colb = (g & 7) * 16
                xs = [xc[d, pl.ds(g16, 16)] for d in range(4)]
                pos = [xs[d] * rm[d] for d in range(4)]
                p0i = [pos[d].astype(jnp.int32) for d in range(4)]
                frac = [pos[d] - p0i[d].astype(jnp.float32) for d in range(4)]
                h0 = [p0i[d] * _PRIMES[d] for d in range(4)]
                h1 = [h0[d] + _PRIMES[d] for d in range(4)]
                H = [h0, h1]
                w1 = frac
                w0 = [1.0 - frac[d] for d in range(4)]
                WD = [w0, w1]
                A = [[H[b0][0] ^ H[b1][1] for b1 in (0, 1)] for b0 in (0, 1)]
                B = [[H[b2][2] ^ H[b3][3] for b3 in (0, 1)] for b2 in (0, 1)]
                WA = [[WD[b0][0] * WD[b1][1] for b1 in (0, 1)] for b0 in (0, 1)]
                WB = [[WD[b2][2] * WD[b3][3] for b3 in (0, 1)] for b2 in (0, 1)]
                for c in range(16):
                    b0, b1, b2, b3 = c & 1, (c >> 1) & 1, (c >> 2) & 1, (c >> 3) & 1
                    idx_c = ((A[b0][b1] ^ B[b2][b3]) & _MASK) + lbase
                    w2 = idx_c + idx_c
                    ibuf[2 * c + kk, pl.ds(colb, 16)] = w2
                    ibuf[_KROWS + 2 * c + kk, pl.ds(colb, 16)] = w2 + 1
                    wbuf[c, pl.ds(g16, 16)] = WA[b0][b1] * WB[b2][b3]
                return c3

            lax.fori_loop(0, _NG, idx_body, 0)

            def fire(k, c4):
                pltpu.async_copy(tbl_hbm.at[ibuf.at[k]], rows.at[k], sem)
                return c4

            lax.fori_loop(0, 2 * _KROWS, fire, 0)

            def drain(k, c5):
                pltpu.make_async_copy(tbl_hbm.at[ibuf.at[0]], rows.at[0], sem).wait()
                return c5

            lax.fori_loop(0, 2 * _KROWS, drain, 0)

            def acc_body(g, c6):
                g16 = g * 16
                kk = g >> 3
                colb = (g & 7) * 16
                acc0 = jnp.zeros((16,), jnp.float32)
                acc1 = jnp.zeros((16,), jnp.float32)
                for c in range(16):
                    w = wbuf[c, pl.ds(g16, 16)]
                    f0 = rows[2 * c + kk, pl.ds(colb, 16)]
                    f1 = rows[_KROWS + 2 * c + kk, pl.ds(colb, 16)]
                    acc0 = acc0 + w * f0
                    acc1 = acc1 + w * f1
                fbuf[2 * l, pl.ds(g16, 16)] = acc0
                fbuf[2 * l + 1, pl.ds(g16, 16)] = acc1
                return c6

            lax.fori_loop(0, _NG, acc_body, 0)
            return carry2

        lax.fori_loop(0, _L, level_body, 0)
        pltpu.sync_copy(fbuf, out_hbm.at[gchunk])
        return carry

    lax.fori_loop(0, _CHUNKS_PER_TILE, chunk_body, 0)


_NCHUNKS = _N // _CHUNK


@functools.partial(jax.jit, static_argnames=())
def _encode(x, tblf, rsp):
    mesh = plsc.VectorSubcoreMesh(core_axis_name="c", subcore_axis_name="s")
    f = pl.kernel(
        _enc_body,
        out_type=jax.ShapeDtypeStruct((_NCHUNKS, _L * _F, _CHUNK), jnp.float32),
        mesh=mesh,
        compiler_params=pltpu.CompilerParams(needs_layout_passes=False),
        scratch_types=[
            pltpu.VMEM((_CHUNK, 4), jnp.float32),
            pltpu.VMEM((4, _CHUNK), jnp.float32),
            pltpu.VMEM((_L, 4, 16), jnp.float32),
            pltpu.VMEM((2 * _KROWS, 128), jnp.int32),
            pltpu.VMEM((16, _CHUNK), jnp.float32),
            pltpu.VMEM((2 * _KROWS, 128), jnp.float32),
            pltpu.VMEM((_L * _F, _CHUNK), jnp.float32),
            pltpu.SemaphoreType.DMA,
        ],
    )
    return f(x, tblf, rsp)


_MLP_CPB = 16


def _mlp_body(x_ref, w0_ref, w1_ref, w2_ref, wo_ref, bo_ref, o_ref):
    x = jnp.concatenate([x_ref[c] for c in range(_MLP_CPB)], axis=1)
    h = jnp.maximum(jnp.dot(w0_ref[...], x, preferred_element_type=jnp.float32), 0.0)
    h = jnp.maximum(jnp.dot(w1_ref[...], h, preferred_element_type=jnp.float32), 0.0)
    h = jnp.maximum(jnp.dot(w2_ref[...], h, preferred_element_type=jnp.float32), 0.0)
    for c in range(_MLP_CPB):
        v = lax.dot_general(
            h[:, c * _CHUNK:(c + 1) * _CHUNK], wo_ref[...],
            (((0,), (0,)), ((), ())), preferred_element_type=jnp.float32)
        o_ref[c] = v + bo_ref[...]


def _mlp(feats, W0T, W1T, W2T, Wout, bout2):
    grid = (_NCHUNKS // _MLP_CPB,)
    return pl.pallas_call(
        _mlp_body,
        grid=grid,
        in_specs=[
            pl.BlockSpec((_MLP_CPB, _L * _F, _CHUNK), lambda i: (i, 0, 0)),
            pl.BlockSpec((_HID, _L * _F), lambda i: (0, 0)),
            pl.BlockSpec((_HID, _HID), lambda i: (0, 0)),
            pl.BlockSpec((_HID, _HID), lambda i: (0, 0)),
            pl.BlockSpec((_HID, 3), lambda i: (0, 0)),
            pl.BlockSpec((1, 3), lambda i: (0, 0)),
        ],
        out_specs=pl.BlockSpec((_MLP_CPB, _CHUNK, 3), lambda i: (i, 0, 0)),
        out_shape=jax.ShapeDtypeStruct((_NCHUNKS, _CHUNK, 3), jnp.float32),
    )(feats, W0T, W1T, W2T, Wout, bout2)


def kernel(x, table, W0, W1, W2, Wout, bout):
    tblf = table.reshape(_L * _T * _F) - bout[0]
    rsp = jnp.asarray(np.broadcast_to(_RM1[:, :, None], (_L, 4, 16)).copy())
    feats = _encode(x, tblf, rsp)
    out = _mlp(feats, W0.T, W1.T, W2.T, Wout, bout.reshape(1, 3))
    return out.reshape(_N, 3)


# speedup vs baseline: 4.4889x; 4.4889x over previous
import functools

import numpy as np
import jax
import jax.numpy as jnp
from jax import lax
from jax.experimental import pallas as pl
from jax.experimental.pallas import tpu as pltpu
from jax.experimental.pallas import tpu_sc as plsc

_L = 16
_F = 2
_T = 2 ** 19
_N = 131072
_HID = 64
_MASK = _T - 1

_min_res = np.array([16.0, 16.0, 16.0, 16.0])
_max_res = np.array([256.0, 256.0, 256.0, 128.0])
_b = np.exp((np.log(_max_res) - np.log(_min_res)) / (_L - 1))
_RES = np.floor(_min_res[None, :] * (_b[None, :] ** np.arange(_L)[:, None])).astype(np.int64)
_RM1 = _RES.astype(np.float32) - 1.0
_PRIMES = [int(np.int32(np.uint32(p))) for p in (1, 2654435761, 805459861, 3674653429)]

_NTILES = 32
_PTS_PER_TILE = _N // _NTILES
_CHUNK = 256
_CHUNKS_PER_TILE = _PTS_PER_TILE // _CHUNK
_NG = _CHUNK // 16
_NROWS = _CHUNK * 16
_KROWS = _NROWS // 128


def _enc_body(x_hbm, tbl_hbm, rsp_hbm, out_hbm, xcr, xc, rs, ibuf, wbuf, rows, fbuf, sem):
    cid = lax.axis_index("c")
    sid = lax.axis_index("s")
    wid = cid * 16 + sid
    pltpu.sync_copy(rsp_hbm, rs)
    iota = lax.iota(jnp.int32, 16)
    zi = jnp.zeros((16,), jnp.int32)

    def chunk_body(ch, carry):
        gchunk = wid * _CHUNKS_PER_TILE + ch
        pt0 = gchunk * _CHUNK
        pltpu.sync_copy(x_hbm.at[pl.ds(pt0, _CHUNK)], xcr)

        def tr_body(g, ct):
            g16 = g * 16
            ridx = iota + g16
            for d in range(4):
                xc[d, pl.ds(g16, 16)] = plsc.load_gather(xcr, [ridx, zi + d])
            return ct

        lax.fori_loop(0, _NG, tr_body, 0)

        def level_body(l, carry2):
            lbase = l * (2 * _T)
            rm = [rs[l, d, :] for d in range(4)]

            def idx_body(g, c3):
                g16 = g * 16
                kk = g >> 3
                colb = (g & 7) * 16
                xs = [xc[d, pl.ds(g16, 16)] for d in range(4)]
                pos = [xs[d] * rm[d] for d in range(4)]
                p0i = [pos[d].astype(jnp.int32) for d in range(4)]
                frac = [pos[d] - p0i[d].astype(jnp.float32) for d in range(4)]
                h0 = [p0i[d] * _PRIMES[d] for d in range(4)]
                h1 = [h0[d] + _PRIMES[d] for d in range(4)]
                H = [h0, h1]
                w1 = frac
                w0 = [1.0 - frac[d] for d in range(4)]
                WD = [w0, w1]
                A = [[H[b0][0] ^ H[b1][1] for b1 in (0, 1)] for b0 in (0, 1)]
                B = [[H[b2][2] ^ H[b3][3] for b3 in (0, 1)] for b2 in (0, 1)]
                WA = [[WD[b0][0] * WD[b1][1] for b1 in (0, 1)] for b0 in (0, 1)]
                WB = [[WD[b2][2] * WD[b3][3] for b3 in (0, 1)] for b2 in (0, 1)]
                for c in range(16):
                    b0, b1, b2, b3 = c & 1, (c >> 1) & 1, (c >> 2) & 1, (c >> 3) & 1
                    idx_c = (A[b0][b1] ^ B[b2][b3]) & _MASK
                    w0 = lbase + ((idx_c >> 7) << 8) + (idx_c & 127)
                    ibuf[2 * c + kk, pl.ds(colb, 16)] = w0
                    ibuf[_KROWS + 2 * c + kk, pl.ds(colb, 16)] = w0 + 128
                    wbuf[c, pl.ds(g16, 16)] = WA[b0][b1] * WB[b2][b3]
                return c3

            lax.fori_loop(0, _NG, idx_body, 0)

            def fire(k, c4):
                pltpu.async_copy(tbl_hbm.at[ibuf.at[k]], rows.at[k], sem)
                return c4

            lax.fori_loop(0, 2 * _KROWS, fire, 0)

            def drain(k, c5):
                pltpu.make_async_copy(tbl_hbm.at[ibuf.at[0]], rows.at[0], sem).wait()
                return c5

            lax.fori_loop(0, 2 * _KROWS, drain, 0)

            def acc_body(g, c6):
                g16 = g * 16
                kk = g >> 3
                colb = (g & 7) * 16
                acc0 = jnp.zeros((16,), jnp.float32)
                acc1 = jnp.zeros((16,), jnp.float32)
                for c in range(16):
                    w = wbuf[c, pl.ds(g16, 16)]
                    f0 = rows[2 * c + kk, pl.ds(colb, 16)]
                    f1 = rows[_KROWS + 2 * c + kk, pl.ds(colb, 16)]
                    acc0 = acc0 + w * f0
                    acc1 = acc1 + w * f1
                fbuf[2 * l, pl.ds(g16, 16)] = acc0
                fbuf[2 * l + 1, pl.ds(g16, 16)] = acc1
                return c6

            lax.fori_loop(0, _NG, acc_body, 0)
            return carry2

        lax.fori_loop(0, _L, level_body, 0)
        pltpu.sync_copy(fbuf, out_hbm.at[gchunk])
        return carry

    lax.fori_loop(0, _CHUNKS_PER_TILE, chunk_body, 0)


_NCHUNKS = _N // _CHUNK


@functools.partial(jax.jit, static_argnames=())
def _encode(x, tblf, rsp):
    mesh = plsc.VectorSubcoreMesh(core_axis_name="c", subcore_axis_name="s")
    f = pl.kernel(
        _enc_body,
        out_type=jax.ShapeDtypeStruct((_NCHUNKS, _L * _F, _CHUNK), jnp.float32),
        mesh=mesh,
        compiler_params=pltpu.CompilerParams(needs_layout_passes=False),
        scratch_types=[
            pltpu.VMEM((_CHUNK, 4), jnp.float32),
            pltpu.VMEM((4, _CHUNK), jnp.float32),
            pltpu.VMEM((_L, 4, 16), jnp.float32),
            pltpu.VMEM((2 * _KROWS, 128), jnp.int32),
            pltpu.VMEM((16, _CHUNK), jnp.float32),
            pltpu.VMEM((2 * _KROWS, 128), jnp.float32),
            pltpu.VMEM((_L * _F, _CHUNK), jnp.float32),
            pltpu.SemaphoreType.DMA,
        ],
    )
    return f(x, tblf, rsp)


_MLP_CPB = 16


def _mlp_body(x_ref, w0_ref, w1_ref, w2_ref, wo_ref, bo_ref, o_ref):
    x = jnp.concatenate([x_ref[c] for c in range(_MLP_CPB)], axis=1)
    h = jnp.maximum(jnp.dot(w0_ref[...], x, preferred_element_type=jnp.float32), 0.0)
    h = jnp.maximum(jnp.dot(w1_ref[...], h, preferred_element_type=jnp.float32), 0.0)
    h = jnp.maximum(jnp.dot(w2_ref[...], h, preferred_element_type=jnp.float32), 0.0)
    for c in range(_MLP_CPB):
        v = lax.dot_general(
            h[:, c * _CHUNK:(c + 1) * _CHUNK], wo_ref[...],
            (((0,), (0,)), ((), ())), preferred_element_type=jnp.float32)
        o_ref[c] = v + bo_ref[...]


def _mlp(feats, W0T, W1T, W2T, Wout, bout2):
    grid = (_NCHUNKS // _MLP_CPB,)
    return pl.pallas_call(
        _mlp_body,
        grid=grid,
        in_specs=[
            pl.BlockSpec((_MLP_CPB, _L * _F, _CHUNK), lambda i: (i, 0, 0)),
            pl.BlockSpec((_HID, _L * _F), lambda i: (0, 0)),
            pl.BlockSpec((_HID, _HID), lambda i: (0, 0)),
            pl.BlockSpec((_HID, _HID), lambda i: (0, 0)),
            pl.BlockSpec((_HID, 3), lambda i: (0, 0)),
            pl.BlockSpec((1, 3), lambda i: (0, 0)),
        ],
        out_specs=pl.BlockSpec((_MLP_CPB, _CHUNK, 3), lambda i: (i, 0, 0)),
        out_shape=jax.ShapeDtypeStruct((_NCHUNKS, _CHUNK, 3), jnp.float32),
    )(feats, W0T, W1T, W2T, Wout, bout2)


def kernel(x, table, W0, W1, W2, Wout, bout):
    tblf = table.reshape(_L, _T // 128, 128, _F).transpose(0, 1, 3, 2).reshape(_L * _T * _F)
    rsp = jnp.asarray(np.broadcast_to(_RM1[:, :, None], (_L, 4, 16)).copy())
    feats = _encode(x, tblf, rsp)
    out = _mlp(feats, W0.T, W1.T, W2.T, Wout, bout.reshape(1, 3))
    return out.reshape(_N, 3)


# speedup vs baseline: 6.5979x; 1.4698x over previous
import functools

import numpy as np
import jax
import jax.numpy as jnp
from jax import lax
from jax.experimental import pallas as pl
from jax.experimental.pallas import tpu as pltpu
from jax.experimental.pallas import tpu_sc as plsc

_L = 16
_F = 2
_T = 2 ** 19
_N = 131072
_HID = 64
_MASK = _T - 1

_min_res = np.array([16.0, 16.0, 16.0, 16.0])
_max_res = np.array([256.0, 256.0, 256.0, 128.0])
_b = np.exp((np.log(_max_res) - np.log(_min_res)) / (_L - 1))
_RES = np.floor(_min_res[None, :] * (_b[None, :] ** np.arange(_L)[:, None])).astype(np.int64)
_RM1 = _RES.astype(np.float32) - 1.0
_PRIMES = [int(np.int32(np.uint32(p))) for p in (1, 2654435761, 805459861, 3674653429)]

_NTILES = 32
_PTS_PER_TILE = _N // _NTILES
_CHUNK = 256
_CHUNKS_PER_TILE = _PTS_PER_TILE // _CHUNK
_NG = _CHUNK // 16
_NROWS = _CHUNK * 16
_NSTREAM = _NROWS // 128
_NCHUNKS = _N // _CHUNK

_PERM = np.zeros((256, 256), dtype=np.float32)
for _j in range(256):
    _PERM[_j, 2 * (_j & 127) + (_j >> 7)] = 1.0

_IL_BLK = 262144


def _il_body(x_ref, p_ref, o_ref):
    x = x_ref[...].reshape(_IL_BLK // 256, 256)
    o = jnp.dot(x, p_ref[...], preferred_element_type=jnp.float32,
                precision=lax.Precision.HIGHEST)
    o_ref[...] = o.reshape(_IL_BLK)


def _interleave(tblf):
    grid = (_L * _T * _F // _IL_BLK,)
    return pl.pallas_call(
        _il_body,
        grid=grid,
        in_specs=[
            pl.BlockSpec((_IL_BLK,), lambda i: (i,)),
            pl.BlockSpec((256, 256), lambda i: (0, 0)),
        ],
        out_specs=pl.BlockSpec((_IL_BLK,), lambda i: (i,)),
        out_shape=jax.ShapeDtypeStruct((_L * _T * _F,), jnp.float32),
    )(tblf, jnp.asarray(_PERM))


def _enc_body(x_hbm, tbl_hbm, rsp_hbm, out_hbm, xcr, xc, rs, ibuf, sbuf, wbuf, rows, fbuf, sem):
    cid = lax.axis_index("c")
    sid = lax.axis_index("s")
    wid = cid * 16 + sid
    pltpu.sync_copy(rsp_hbm, rs)
    iota = lax.iota(jnp.int32, 16)
    zi = jnp.zeros((16,), jnp.int32)
    iota_h = iota >> 1
    iota_o4 = (iota & 1) * 4

    def chunk_body(ch, carry):
        gchunk = wid * _CHUNKS_PER_TILE + ch
        pt0 = gchunk * _CHUNK
        pltpu.sync_copy(x_hbm.at[pl.ds(pt0 // 2, _CHUNK // 2)], xcr)

        def tr_body(g, ct):
            g16 = g * 16
            vrow = iota_h + g * 8
            for d in range(4):
                xc[d, pl.ds(g16, 16)] = plsc.load_gather(xcr, [vrow, iota_o4 + d])
            return ct

        lax.fori_loop(0, _NG, tr_body, 0)

        def level_body(l, carry2):
            lbase = l * _T
            rm = [rs[l, d, :] for d in range(4)]

            def idx_body(g, c3):
                g16 = g * 16
                kk = g >> 3
                colb = (g & 7) * 16
                xs = [xc[d, pl.ds(g16, 16)] for d in range(4)]
                pos = [xs[d] * rm[d] for d in range(4)]
                p0i = [pos[d].astype(jnp.int32) for d in range(4)]
                frac = [pos[d] - p0i[d].astype(jnp.float32) for d in range(4)]
                h0 = [p0i[d] * _PRIMES[d] for d in range(4)]
                h1 = [h0[d] + _PRIMES[d] for d in range(4)]
                H = [h0, h1]
                w1 = frac
                w0 = [1.0 - frac[d] for d in range(4)]
                WD = [w0, w1]
                A = [[H[b0][0] ^ H[b1][1] for b1 in (0, 1)] for b0 in (0, 1)]
                B = [[H[b2][2] ^ H[b3][3] for b3 in (0, 1)] for b2 in (0, 1)]
                WA = [[WD[b0][0] * WD[b1][1] for b1 in (0, 1)] for b0 in (0, 1)]
                WB = [[WD[b2][2] * WD[b3][3] for b3 in (0, 1)] for b2 in (0, 1)]
                for c in range(16):
                    b0, b1, b2, b3 = c & 1, (c >> 1) & 1, (c >> 2) & 1, (c >> 3) & 1
                    s_c = ((A[b0][b1] ^ B[b2][b3]) & _MASK) + lbase
                    ibuf[2 * c + kk, pl.ds(colb, 16)] = s_c >> 2
                    sbuf[c, pl.ds(g16, 16)] = (s_c & 3) * 2
                    wbuf[c, pl.ds(g16, 16)] = WA[b0][b1] * WB[b2][b3]
                return c3

            lax.fori_loop(0, _NG, idx_body, 0)

            def fire(k, c4):
                pltpu.async_copy(tbl_hbm.at[ibuf.at[k]], rows.at[k], sem)
                return c4

            lax.fori_loop(0, _NSTREAM, fire, 0)

            def drain(k, c5):
                pltpu.make_async_copy(tbl_hbm.at[ibuf.at[0]], rows.at[0], sem).wait()
                return c5

            lax.fori_loop(0, _NSTREAM, drain, 0)

            def acc_body(g, c6):
                g16 = g * 16
                kk = g >> 3
                colb = (g & 7) * 16
                vcol = iota + colb
                vrow0 = zi + kk
                acc0 = jnp.zeros((16,), jnp.float32)
                acc1 = jnp.zeros((16,), jnp.float32)
                for c in range(16):
                    vrow = vrow0 + 2 * c
                    vsub = sbuf[c, pl.ds(g16, 16)]
                    w = wbuf[c, pl.ds(g16, 16)]
                    f0 = plsc.load_gather(rows, [vrow, vcol, vsub])
                    f1 = plsc.load_gather(rows, [vrow, vcol, vsub + 1])
                    acc0 = acc0 + w * f0
                    acc1 = acc1 + w * f1
                fbuf[2 * l, pl.ds(g16, 16)] = acc0
                fbuf[2 * l + 1, pl.ds(g16, 16)] = acc1
                return c6

            lax.fori_loop(0, _NG, acc_body, 0)
            return carry2

        lax.fori_loop(0, _L, level_body, 0)
        pltpu.sync_copy(fbuf, out_hbm.at[gchunk])
        return carry

    lax.fori_loop(0, _CHUNKS_PER_TILE, chunk_body, 0)


@functools.partial(jax.jit, static_argnames=())
def _encode(x, tbl2, rsp):
    mesh = plsc.VectorSubcoreMesh(core_axis_name="c", subcore_axis_name="s")
    f = pl.kernel(
        _enc_body,
        out_type=jax.ShapeDtypeStruct((_NCHUNKS, _L * _F, _CHUNK), jnp.float32),
        mesh=mesh,
        compiler_params=pltpu.CompilerParams(
            needs_layout_passes=False, use_tc_tiling_on_sc=False),
        scratch_types=[
            pltpu.VMEM((_CHUNK // 2, 8), jnp.float32),
            pltpu.VMEM((4, _CHUNK), jnp.float32),
            pltpu.VMEM((_L, 4, 16), jnp.float32),
            pltpu.VMEM((_NSTREAM, 128), jnp.int32),
            pltpu.VMEM((16, _CHUNK), jnp.int32),
            pltpu.VMEM((16, _CHUNK), jnp.float32),
            pltpu.VMEM((_NSTREAM, 128, 8), jnp.float32),
            pltpu.VMEM((_L * _F, _CHUNK), jnp.float32),
            pltpu.SemaphoreType.DMA,
        ],
    )
    return f(x, tbl2, rsp)


_MLP_CPB = 16


def _mlp_body(x_ref, w0_ref, w1_ref, w2_ref, wo_ref, bo_ref, o_ref):
    x = jnp.concatenate([x_ref[c] for c in range(_MLP_CPB)], axis=1)
    h = jnp.maximum(jnp.dot(w0_ref[...], x, preferred_element_type=jnp.float32), 0.0)
    h = jnp.maximum(jnp.dot(w1_ref[...], h, preferred_element_type=jnp.float32), 0.0)
    h = jnp.maximum(jnp.dot(w2_ref[...], h, preferred_element_type=jnp.float32), 0.0)
    for c in range(_MLP_CPB):
        v = lax.dot_general(
            h[:, c * _CHUNK:(c + 1) * _CHUNK], wo_ref[...],
            (((0,), (0,)), ((), ())), preferred_element_type=jnp.float32)
        o_ref[c] = v + bo_ref[...]


def _mlp(feats, W0T, W1T, W2T, Wout, bout2):
    grid = (_NCHUNKS // _MLP_CPB,)
    return pl.pallas_call(
        _mlp_body,
        grid=grid,
        in_specs=[
            pl.BlockSpec((_MLP_CPB, _L * _F, _CHUNK), lambda i: (i, 0, 0)),
            pl.BlockSpec((_HID, _L * _F), lambda i: (0, 0)),
            pl.BlockSpec((_HID, _HID), lambda i: (0, 0)),
            pl.BlockSpec((_HID, _HID), lambda i: (0, 0)),
            pl.BlockSpec((_HID, 3), lambda i: (0, 0)),
            pl.BlockSpec((1, 3), lambda i: (0, 0)),
        ],
        out_specs=pl.BlockSpec((_MLP_CPB, _CHUNK, 3), lambda i: (i, 0, 0)),
        out_shape=jax.ShapeDtypeStruct((_NCHUNKS, _CHUNK, 3), jnp.float32),
    )(feats, W0T, W1T, W2T, Wout, bout2)


def kernel(x, table, W0, W1, W2, Wout, bout):
    tblf = table.reshape(_L, _T // 128, 128, _F).transpose(0, 1, 3, 2).reshape(_L * _T * _F)
    tbli = _interleave(tblf)
    tbl8 = tbli.reshape(_L * _T * _F // 8, 8)
    rsp = jnp.asarray(np.broadcast_to(_RM1[:, :, None], (_L, 4, 16)).copy())
    feats = _encode(x.reshape(_N // 2, 8), tbl8, rsp)
    out = _mlp(feats, W0.T, W1.T, W2.T, Wout, bout.reshape(1, 3))
    return out.reshape(_N, 3)


# speedup vs baseline: 9.0846x; 1.3769x over previous
import functools

import numpy as np
import jax
import jax.numpy as jnp
from jax import lax
from jax.experimental import pallas as pl
from jax.experimental.pallas import tpu as pltpu
from jax.experimental.pallas import tpu_sc as plsc

_L = 16
_F = 2
_T = 2 ** 19
_N = 131072
_HID = 64
_MASK = _T - 1

_min_res = np.array([16.0, 16.0, 16.0, 16.0])
_max_res = np.array([256.0, 256.0, 256.0, 128.0])
_b = np.exp((np.log(_max_res) - np.log(_min_res)) / (_L - 1))
_RES = np.floor(_min_res[None, :] * (_b[None, :] ** np.arange(_L)[:, None])).astype(np.int64)
_RM1 = _RES.astype(np.float32) - 1.0
_PRIMES = [int(np.int32(np.uint32(p))) for p in (1, 2654435761, 805459861, 3674653429)]

_NTILES = 32
_PTS_PER_TILE = _N // _NTILES
_CHUNK = 256
_CHUNKS_PER_TILE = _PTS_PER_TILE // _CHUNK
_NG = _CHUNK // 16
_NROWS = _CHUNK * 16
_NSTREAM = _NROWS // 128
_NCHUNKS = _N // _CHUNK

_PERM = np.zeros((256, 256), dtype=np.float32)
for _j in range(256):
    _PERM[_j, 2 * (_j & 127) + (_j >> 7)] = 1.0

_IL_BLK = 262144


def _il_body(x_ref, p_ref, o_ref):
    x = x_ref[...].reshape(_IL_BLK // 256, 256)
    o = jnp.dot(x, p_ref[...], preferred_element_type=jnp.float32,
                precision=lax.Precision.HIGHEST)
    o_ref[...] = o.reshape(_IL_BLK)


def _interleave(tblf):
    grid = (_L * _T * _F // _IL_BLK,)
    return pl.pallas_call(
        _il_body,
        grid=grid,
        in_specs=[
            pl.BlockSpec((_IL_BLK,), lambda i: (i,)),
            pl.BlockSpec((256, 256), lambda i: (0, 0)),
        ],
        out_specs=pl.BlockSpec((_IL_BLK,), lambda i: (i,)),
        out_shape=jax.ShapeDtypeStruct((_L * _T * _F,), jnp.float32),
    )(tblf, jnp.asarray(_PERM))


def _enc_body(x_hbm, tbl_hbm, rsp_hbm, out_hbm, xcr, xc, rs, ibuf, sbuf, wbuf, rows, fbuf, sem):
    cid = lax.axis_index("c")
    sid = lax.axis_index("s")
    wid = cid * 16 + sid
    pltpu.sync_copy(rsp_hbm, rs)
    iota = lax.iota(jnp.int32, 16)
    zi = jnp.zeros((16,), jnp.int32)
    iota_h = iota >> 1
    iota_o4 = (iota & 1) * 4

    def chunk_body(ch, carry):
        gchunk = wid * _CHUNKS_PER_TILE + ch
        pt0 = gchunk * _CHUNK
        pltpu.sync_copy(x_hbm.at[pl.ds(pt0 // 2, _CHUNK // 2)], xcr)

        def tr_body(g, ct):
            g16 = g * 16
            vrow = iota_h + g * 8
            for d in range(4):
                xc[d, pl.ds(g16, 16)] = plsc.load_gather(xcr, [vrow, iota_o4 + d])
            return ct

        lax.fori_loop(0, _NG, tr_body, 0)

        def compute_and_fire(lv, b):
            lbase = lv * _T
            rm = [rs[lv, d, :] for d in range(4)]

            def idx_body(g, c3):
                g16 = g * 16
                kk = g >> 3
                colb = (g & 7) * 16
                xs = [xc[d, pl.ds(g16, 16)] for d in range(4)]
                pos = [xs[d] * rm[d] for d in range(4)]
                p0i = [pos[d].astype(jnp.int32) for d in range(4)]
                frac = [pos[d] - p0i[d].astype(jnp.float32) for d in range(4)]
                h0 = [p0i[d] * _PRIMES[d] for d in range(4)]
                h1 = [h0[d] + _PRIMES[d] for d in range(4)]
                H = [h0, h1]
                w1 = frac
                w0 = [1.0 - frac[d] for d in range(4)]
                WD = [w0, w1]
                A = [[H[b0][0] ^ H[b1][1] for b1 in (0, 1)] for b0 in (0, 1)]
                B = [[H[b2][2] ^ H[b3][3] for b3 in (0, 1)] for b2 in (0, 1)]
                WA = [[WD[b0][0] * WD[b1][1] for b1 in (0, 1)] for b0 in (0, 1)]
                WB = [[WD[b2][2] * WD[b3][3] for b3 in (0, 1)] for b2 in (0, 1)]
                for c in range(16):
                    b0, b1, b2, b3 = c & 1, (c >> 1) & 1, (c >> 2) & 1, (c >> 3) & 1
                    s_c = ((A[b0][b1] ^ B[b2][b3]) & _MASK) + lbase
                    ibuf[b, 2 * c + kk, pl.ds(colb, 16)] = s_c >> 2
                    sbuf[b, c, pl.ds(g16, 16)] = (s_c & 3) * 2
                    wbuf[b, c, pl.ds(g16, 16)] = WA[b0][b1] * WB[b2][b3]
                return c3

            lax.fori_loop(0, _NG, idx_body, 0)

            def fire(k, c4):
                pltpu.async_copy(tbl_hbm.at[ibuf.at[b, k]], rows.at[b, k], sem.at[b])
                return c4

            lax.fori_loop(0, _NSTREAM, fire, 0)

        def drain_and_acc(l, b):
            def drain(k, c5):
                pltpu.make_async_copy(
                    tbl_hbm.at[ibuf.at[0, 0]], rows.at[0, 0], sem.at[b]).wait()
                return c5

            lax.fori_loop(0, _NSTREAM, drain, 0)
            vb = zi + b

            def acc_body(g, c6):
                g16 = g * 16
                kk = g >> 3
                colb = (g & 7) * 16
                vcol = iota + colb
                vrow0 = zi + kk
                acc0 = jnp.zeros((16,), jnp.float32)
                acc1 = jnp.zeros((16,), jnp.float32)
                for c in range(16):
                    vrow = vrow0 + 2 * c
                    vsub = sbuf[b, c, pl.ds(g16, 16)]
                    w = wbuf[b, c, pl.ds(g16, 16)]
                    f0 = plsc.load_gather(rows, [vb, vrow, vcol, vsub])
                    f1 = plsc.load_gather(rows, [vb, vrow, vcol, vsub + 1])
                    acc0 = acc0 + w * f0
                    acc1 = acc1 + w * f1
                fbuf[2 * l, pl.ds(g16, 16)] = acc0
                fbuf[2 * l + 1, pl.ds(g16, 16)] = acc1
                return c6

            lax.fori_loop(0, _NG, acc_body, 0)

        compute_and_fire(0, 0)

        def level_body(l, carry2):
            b = l & 1

            @pl.when(l + 1 < _L)
            def _():
                compute_and_fire(l + 1, 1 - b)

            drain_and_acc(l, b)
            return carry2

        lax.fori_loop(0, _L, level_body, 0)
        pltpu.sync_copy(fbuf, out_hbm.at[gchunk])
        return carry

    lax.fori_loop(0, _CHUNKS_PER_TILE, chunk_body, 0)


@functools.partial(jax.jit, static_argnames=())
def _encode(x, tbl2, rsp):
    mesh = plsc.VectorSubcoreMesh(core_axis_name="c", subcore_axis_name="s")
    f = pl.kernel(
        _enc_body,
        out_type=jax.ShapeDtypeStruct((_NCHUNKS, _L * _F, _CHUNK), jnp.float32),
        mesh=mesh,
        compiler_params=pltpu.CompilerParams(
            needs_layout_passes=False, use_tc_tiling_on_sc=False),
        scratch_types=[
            pltpu.VMEM((_CHUNK // 2, 8), jnp.float32),
            pltpu.VMEM((4, _CHUNK), jnp.float32),
            pltpu.VMEM((_L, 4, 16), jnp.float32),
            pltpu.VMEM((2, _NSTREAM, 128), jnp.int32),
            pltpu.VMEM((2, 16, _CHUNK), jnp.int32),
            pltpu.VMEM((2, 16, _CHUNK), jnp.float32),
            pltpu.VMEM((2, _NSTREAM, 128, 8), jnp.float32),
            pltpu.VMEM((_L * _F, _CHUNK), jnp.float32),
            pltpu.SemaphoreType.DMA((2,)),
        ],
    )
    return f(x, tbl2, rsp)


_MLP_CPB = 16


def _mlp_body(x_ref, w0_ref, w1_ref, w2_ref, wo_ref, bo_ref, o_ref):
    x = jnp.concatenate([x_ref[c] for c in range(_MLP_CPB)], axis=1)
    h = jnp.maximum(jnp.dot(w0_ref[...], x, preferred_element_type=jnp.float32), 0.0)
    h = jnp.maximum(jnp.dot(w1_ref[...], h, preferred_element_type=jnp.float32), 0.0)
    h = jnp.maximum(jnp.dot(w2_ref[...], h, preferred_element_type=jnp.float32), 0.0)
    for c in range(_MLP_CPB):
        v = lax.dot_general(
            h[:, c * _CHUNK:(c + 1) * _CHUNK], wo_ref[...],
            (((0,), (0,)), ((), ())), preferred_element_type=jnp.float32)
        o_ref[c] = v + bo_ref[...]


def _mlp(feats, W0T, W1T, W2T, Wout, bout2):
    grid = (_NCHUNKS // _MLP_CPB,)
    return pl.pallas_call(
        _mlp_body,
        grid=grid,
        in_specs=[
            pl.BlockSpec((_MLP_CPB, _L * _F, _CHUNK), lambda i: (i, 0, 0)),
            pl.BlockSpec((_HID, _L * _F), lambda i: (0, 0)),
            pl.BlockSpec((_HID, _HID), lambda i: (0, 0)),
            pl.BlockSpec((_HID, _HID), lambda i: (0, 0)),
            pl.BlockSpec((_HID, 3), lambda i: (0, 0)),
            pl.BlockSpec((1, 3), lambda i: (0, 0)),
        ],
        out_specs=pl.BlockSpec((_MLP_CPB, _CHUNK, 3), lambda i: (i, 0, 0)),
        out_shape=jax.ShapeDtypeStruct((_NCHUNKS, _CHUNK, 3), jnp.float32),
    )(feats, W0T, W1T, W2T, Wout, bout2)


def kernel(x, table, W0, W1, W2, Wout, bout):
    tblf = table.reshape(_L, _T // 128, 128, _F).transpose(0, 1, 3, 2).reshape(_L * _T * _F)
    tbli = _interleave(tblf)
    tbl8 = tbli.reshape(_L * _T * _F // 8, 8)
    rsp = jnp.asarray(np.broadcast_to(_RM1[:, :, None], (_L, 4, 16)).copy())
    feats = _encode(x.reshape(_N // 2, 8), tbl8, rsp)
    out = _mlp(feats, W0.T, W1.T, W2.T, Wout, bout.reshape(1, 3))
    return out.reshape(_N, 3)


# speedup vs baseline: 9.2409x; 1.0172x over previous
import functools

import numpy as np
import jax
import jax.numpy as jnp
from jax import lax
from jax.experimental import pallas as pl
from jax.experimental.pallas import tpu as pltpu
from jax.experimental.pallas import tpu_sc as plsc

_L = 16
_F = 2
_T = 2 ** 19
_N = 131072
_HID = 64
_MASK = _T - 1

_min_res = np.array([16.0, 16.0, 16.0, 16.0])
_max_res = np.array([256.0, 256.0, 256.0, 128.0])
_b = np.exp((np.log(_max_res) - np.log(_min_res)) / (_L - 1))
_RES = np.floor(_min_res[None, :] * (_b[None, :] ** np.arange(_L)[:, None])).astype(np.int64)
_RM1 = _RES.astype(np.float32) - 1.0
_PRIMES = [int(np.int32(np.uint32(p))) for p in (1, 2654435761, 805459861, 3674653429)]

_NTILES = 32
_PTS_PER_TILE = _N // _NTILES
_CHUNK = 256
_CHUNKS_PER_TILE = _PTS_PER_TILE // _CHUNK
_NG = _CHUNK // 16
_NROWS = _CHUNK * 16
_NSTREAM = _NROWS // 128
_NCHUNKS = _N // _CHUNK

_PERM = np.zeros((256, 256), dtype=np.float32)
for _j in range(256):
    _PERM[_j, 2 * (_j & 127) + (_j >> 7)] = 1.0

_IL_BLK = 1048576


def _il_body(x_ref, p_ref, o_ref):
    x = x_ref[...].reshape(_IL_BLK // 256, 256)
    o = jnp.dot(x, p_ref[...], preferred_element_type=jnp.float32,
                precision=lax.Precision.HIGHEST)
    o_ref[...] = o.reshape(_IL_BLK)


def _interleave(tblf):
    grid = (_L * _T * _F // _IL_BLK,)
    return pl.pallas_call(
        _il_body,
        grid=grid,
        in_specs=[
            pl.BlockSpec((_IL_BLK,), lambda i: (i,)),
            pl.BlockSpec((256, 256), lambda i: (0, 0)),
        ],
        out_specs=pl.BlockSpec((_IL_BLK,), lambda i: (i,)),
        out_shape=jax.ShapeDtypeStruct((_L * _T * _F,), jnp.float32),
    )(tblf, jnp.asarray(_PERM))


def _enc_body(x_hbm, tbl_hbm, rsp_hbm, out_hbm, xcr, xc, rs, ibuf, sbuf, wbuf, rows, fbuf, sem):
    cid = lax.axis_index("c")
    sid = lax.axis_index("s")
    wid = cid * 16 + sid
    pltpu.sync_copy(rsp_hbm, rs)
    iota = lax.iota(jnp.int32, 16)
    zi = jnp.zeros((16,), jnp.int32)
    iota_h = iota >> 1
    iota_o4 = (iota & 1) * 4

    def chunk_body(ch, carry):
        gchunk = wid * _CHUNKS_PER_TILE + ch
        pt0 = gchunk * _CHUNK
        pltpu.sync_copy(x_hbm.at[pl.ds(pt0 // 2, _CHUNK // 2)], xcr)

        def tr_body(g, ct):
            g16 = g * 16
            vrow = iota_h + g * 8
            for d in range(4):
                xc[d, pl.ds(g16, 16)] = plsc.load_gather(xcr, [vrow, iota_o4 + d])
            return ct

        lax.fori_loop(0, _NG, tr_body, 0)

        def compute_and_fire(lv, b):
            lbase = lv * _T
            rm = [rs[lv, d, :] for d in range(4)]

            def idx_body(g, c3):
                g16 = g * 16
                kk = g >> 3
                colb = (g & 7) * 16
                xs = [xc[d, pl.ds(g16, 16)] for d in range(4)]
                pos = [xs[d] * rm[d] for d in range(4)]
                p0i = [pos[d].astype(jnp.int32) for d in range(4)]
                frac = [pos[d] - p0i[d].astype(jnp.float32) for d in range(4)]
                h0 = [p0i[d] * _PRIMES[d] for d in range(4)]
                h1 = [h0[d] + _PRIMES[d] for d in range(4)]
                H = [h0, h1]
                w1 = frac
                w0 = [1.0 - frac[d] for d in range(4)]
                WD = [w0, w1]
                A = [[H[b0][0] ^ H[b1][1] for b1 in (0, 1)] for b0 in (0, 1)]
                B = [[H[b2][2] ^ H[b3][3] for b3 in (0, 1)] for b2 in (0, 1)]
                WA = [[WD[b0][0] * WD[b1][1] for b1 in (0, 1)] for b0 in (0, 1)]
                WB = [[WD[b2][2] * WD[b3][3] for b3 in (0, 1)] for b2 in (0, 1)]
                for c in range(16):
                    b0, b1, b2, b3 = c & 1, (c >> 1) & 1, (c >> 2) & 1, (c >> 3) & 1
                    s_c = ((A[b0][b1] ^ B[b2][b3]) & _MASK) + lbase
                    ibuf[b, 2 * c + kk, pl.ds(colb, 16)] = s_c >> 2
                    sbuf[b, c, pl.ds(g16, 16)] = (s_c & 3) * 2
                    wbuf[b, c, pl.ds(g16, 16)] = WA[b0][b1] * WB[b2][b3]
                return c3

            lax.fori_loop(0, _NG, idx_body, 0)

            def fire(k, c4):
                pltpu.async_copy(tbl_hbm.at[ibuf.at[b, k]], rows.at[b, k], sem.at[b])
                return c4

            lax.fori_loop(0, _NSTREAM, fire, 0)

        def drain_and_acc(l, b):
            def drain(k, c5):
                pltpu.make_async_copy(
                    tbl_hbm.at[ibuf.at[0, 0]], rows.at[0, 0], sem.at[b]).wait()
                return c5

            lax.fori_loop(0, _NSTREAM, drain, 0)
            vb = zi + b

            def acc_body(g, c6):
                g16 = g * 16
                kk = g >> 3
                colb = (g & 7) * 16
                vcol = iota + colb
                vrow0 = zi + kk
                acc0 = jnp.zeros((16,), jnp.float32)
                acc1 = jnp.zeros((16,), jnp.float32)
                for c in range(16):
                    vrow = vrow0 + 2 * c
                    vsub = sbuf[b, c, pl.ds(g16, 16)]
                    w = wbuf[b, c, pl.ds(g16, 16)]
                    f0 = plsc.load_gather(rows, [vb, vrow, vcol, vsub])
                    f1 = plsc.load_gather(rows, [vb, vrow, vcol, vsub + 1])
                    acc0 = acc0 + w * f0
                    acc1 = acc1 + w * f1
                fbuf[2 * l, pl.ds(g16, 16)] = acc0
                fbuf[2 * l + 1, pl.ds(g16, 16)] = acc1
                return c6

            lax.fori_loop(0, _NG, acc_body, 0)

        compute_and_fire(0, 0)

        def level_body(l, carry2):
            b = l & 1

            @pl.when(l + 1 < _L)
            def _():
                compute_and_fire(l + 1, 1 - b)

            drain_and_acc(l, b)
            return carry2

        lax.fori_loop(0, _L, level_body, 0)
        pltpu.sync_copy(fbuf, out_hbm.at[gchunk])
        return carry

    lax.fori_loop(0, _CHUNKS_PER_TILE, chunk_body, 0)


@functools.partial(jax.jit, static_argnames=())
def _encode(x, tbl2, rsp):
    mesh = plsc.VectorSubcoreMesh(core_axis_name="c", subcore_axis_name="s")
    f = pl.kernel(
        _enc_body,
        out_type=jax.ShapeDtypeStruct((_NCHUNKS, _L * _F, _CHUNK), jnp.float32),
        mesh=mesh,
        compiler_params=pltpu.CompilerParams(
            needs_layout_passes=False, use_tc_tiling_on_sc=False),
        scratch_types=[
            pltpu.VMEM((_CHUNK // 2, 8), jnp.float32),
            pltpu.VMEM((4, _CHUNK), jnp.float32),
            pltpu.VMEM((_L, 4, 16), jnp.float32),
            pltpu.VMEM((2, _NSTREAM, 128), jnp.int32),
            pltpu.VMEM((2, 16, _CHUNK), jnp.int32),
            pltpu.VMEM((2, 16, _CHUNK), jnp.float32),
            pltpu.VMEM((2, _NSTREAM, 128, 8), jnp.float32),
            pltpu.VMEM((_L * _F, _CHUNK), jnp.float32),
            pltpu.SemaphoreType.DMA((2,)),
        ],
    )
    return f(x, tbl2, rsp)


_MLP_CPB = 16


def _mlp_body(x_ref, w0_ref, w1_ref, w2_ref, wo_ref, bo_ref, o_ref):
    x = jnp.concatenate([x_ref[c] for c in range(_MLP_CPB)], axis=1)
    h = jnp.maximum(jnp.dot(w0_ref[...], x, preferred_element_type=jnp.float32), 0.0)
    h = jnp.maximum(jnp.dot(w1_ref[...], h, preferred_element_type=jnp.float32), 0.0)
    h = jnp.maximum(jnp.dot(w2_ref[...], h, preferred_element_type=jnp.float32), 0.0)
    for c in range(_MLP_CPB):
        v = lax.dot_general(
            h[:, c * _CHUNK:(c + 1) * _CHUNK], wo_ref[...],
            (((0,), (0,)), ((), ())), preferred_element_type=jnp.float32)
        o_ref[c] = v + bo_ref[...]


def _mlp(feats, W0T, W1T, W2T, Wout, bout2):
    grid = (_NCHUNKS // _MLP_CPB,)
    return pl.pallas_call(
        _mlp_body,
        grid=grid,
        in_specs=[
            pl.BlockSpec((_MLP_CPB, _L * _F, _CHUNK), lambda i: (i, 0, 0)),
            pl.BlockSpec((_HID, _L * _F), lambda i: (0, 0)),
            pl.BlockSpec((_HID, _HID), lambda i: (0, 0)),
            pl.BlockSpec((_HID, _HID), lambda i: (0, 0)),
            pl.BlockSpec((_HID, 3), lambda i: (0, 0)),
            pl.BlockSpec((1, 3), lambda i: (0, 0)),
        ],
        out_specs=pl.BlockSpec((_MLP_CPB, _CHUNK, 3), lambda i: (i, 0, 0)),
        out_shape=jax.ShapeDtypeStruct((_NCHUNKS, _CHUNK, 3), jnp.float32),
    )(feats, W0T, W1T, W2T, Wout, bout2)


def kernel(x, table, W0, W1, W2, Wout, bout):
    tblf = table.reshape(_L, _T // 128, 128, _F).transpose(0, 1, 3, 2).reshape(_L * _T * _F)
    tbli = _interleave(tblf)
    tbl8 = tbli.reshape(_L * _T * _F // 8, 8)
    rsp = jnp.asarray(np.broadcast_to(_RM1[:, :, None], (_L, 4, 16)).copy())
    feats = _encode(x.reshape(_N // 2, 8), tbl8, rsp)
    out = _mlp(feats, W0.T, W1.T, W2.T, Wout, bout.reshape(1, 3))
    return out.reshape(_N, 3)
